# baseline (speedup 1.0000x reference)
import functools

import jax
import jax.numpy as jnp
from jax import lax
from jax.experimental import pallas as pl
from jax.experimental.pallas import tpu as pltpu
from jax.experimental.pallas import tpu_sc as plsc

N = 2048
NUM_WORKERS = 32
GR = 8
GROUPS = 8
RB = 512
NRB = N // RB
TRIL_BLOCKS = NRB * (NRB + 1) // 2
ANA_BUF = 16576
ANA_LEN = N * (N - 1) // 2
NEG = -1e30

FALSE_NEW = 1.2
FALSE_LINK = 0.5
WRONG_LINK = 1.0


def _blk(g):
    r = ((g >= 1).astype(jnp.int32) + (g >= 3).astype(jnp.int32)
         + (g >= 6).astype(jnp.int32))
    c = g - (r * (r + 1)) // 2
    return r, c


def _tc_ante_body(mask_ref, ante_ref, gidx_ref):
    g = pl.program_id(0)
    rblk, cblk = _blk(g)
    colv = (lax.broadcasted_iota(jnp.int32, (RB, RB), 1) + cblk * RB).astype(jnp.float32)
    part = jnp.sum(mask_ref[...] * colv, axis=1, keepdims=True)

    @pl.when(cblk == 0)
    def _():
        ante_ref[...] = part

    @pl.when(cblk > 0)
    def _():
        ante_ref[...] = ante_ref[...] + part

    @pl.when(cblk == rblk)
    def _():
        rowv = (lax.broadcasted_iota(jnp.int32, (RB, 1), 0) + rblk * RB).astype(jnp.float32)
        tri = rowv * (rowv - 1.0) * 0.5
        gidx_ref[...] = (tri + ante_ref[...]).astype(jnp.int32)


def _sc_body(ana_hbm, gidx_hbm, out_hbm, idx_t, a0, a1, row8, sa0, sa1):
    c = lax.axis_index("c")
    s = lax.axis_index("s")
    wid = s * 2 + c

    iota = lax.iota(jnp.int32, 16)
    negs16 = jnp.full((16,), NEG, jnp.float32)
    abufs, asems = (a0, a1), (sa0, sa1)

    def group_info(g):
        w = wid if g % 2 == 0 else 31 - wid
        i0 = GR * w + 256 * g
        alen = 2048 * (g + 1)
        start = (i0 * (i0 - 1)) // 2
        start8 = jnp.minimum((start // 8) * 8, ANA_LEN - alen)
        return i0, alen, start8

    for g in range(GROUPS):
        i0, _, _ = group_info(g)
        pltpu.sync_copy(gidx_hbm.at[pl.ds(i0, GR)], idx_t.at[pl.ds(GR * g, GR)])

    def issue(g, p):
        i0, alen, start8 = group_info(g)
        ca = pltpu.async_copy(
            ana_hbm.at[pl.ds(start8, alen)],
            abufs[p].at[pl.ds(0, alen)], asems[p])
        return ca, i0, start8

    pend = issue(0, 0)
    for g in range(GROUPS):
        p = g & 1
        ca, i0, start8 = pend
        ca.wait()
        if g + 1 < GROUPS:
            pend = issue(g + 1, 1 - p)
        ab = abufs[p]
        for k in reversed(range(GR)):
            i = i0 + k
            off = (i * (i - 1)) // 2 - start8
            r = GR * g + k
            q = idx_t[pl.ds((r // 16) * 16, 16)][r % 16] - start8

            bvec = ab[pl.ds(q, 16)]
            ab[pl.ds(q, 16)] = jnp.where(iota == 0, NEG, bvec)
            for h in range(8):
                ab[pl.ds(off + i + 16 * h, 16)] = negs16

            trip = (i + 127) // 128

            def col_body(j, mmax, off=off, ab=ab):
                for h in range(8):
                    mmax = jnp.maximum(mmax, ab[pl.ds(off + j * 128 + h * 16, 16)])
                return mmax

            mmax = lax.fori_loop(0, trip, col_body, negs16)

            row8[k, pl.ds(0, 16)] = mmax
            row8[k, pl.ds(16, 16)] = bvec
        pltpu.sync_copy(row8, out_hbm.at[pl.ds(i0, GR)])


def _tc_comb_body(buf_ref, ante_ref, eps_ref, out_ref):
    m = jnp.max(buf_ref[:, 0:16], axis=1, keepdims=True)
    bg = buf_ref[:, 16:17]
    ante = ante_ref[...]
    ei = eps_ref[...]
    rowv = lax.broadcasted_iota(jnp.int32, (N, 1), 0).astype(jnp.float32)
    non_ana = ante == rowv
    b = jnp.where(non_ana, ei, bg)
    cc = jnp.where(non_ana, FALSE_LINK, WRONG_LINK)
    dd = jnp.where(non_ana, NEG, FALSE_NEW * (1.0 + ei - b))
    tt = cc * (1.0 + m - b)
    rl = jnp.maximum(0.0, jnp.maximum(dd, tt))
    out_ref[...] = jnp.broadcast_to(jnp.sum(rl), (1, 1))


@jax.jit
def _run(eps_scores, ana_scores, solution_mask):
    antef, gidx = pl.pallas_call(
        _tc_ante_body,
        grid=(TRIL_BLOCKS,),
        in_specs=[pl.BlockSpec((RB, RB), lambda g: _blk(g))],
        out_specs=[pl.BlockSpec((RB, 1), lambda g: (_blk(g)[0], 0)),
                   pl.BlockSpec((RB, 1), lambda g: (_blk(g)[0], 0))],
        out_shape=[jax.ShapeDtypeStruct((N, 1), jnp.float32),
                   jax.ShapeDtypeStruct((N, 1), jnp.int32)],
    )(solution_mask)

    mesh = plsc.VectorSubcoreMesh(core_axis_name="c", subcore_axis_name="s")
    call = functools.partial(
        pl.kernel,
        mesh=mesh,
        compiler_params=pltpu.CompilerParams(use_tc_tiling_on_sc=False),
        out_type=jax.ShapeDtypeStruct((N, 32), jnp.float32),
        scratch_types=[
            pltpu.VMEM((64,), jnp.int32),
            pltpu.VMEM((ANA_BUF,), jnp.float32),
            pltpu.VMEM((ANA_BUF,), jnp.float32),
            pltpu.VMEM((GR, 32), jnp.float32),
            pltpu.SemaphoreType.DMA,
            pltpu.SemaphoreType.DMA,
        ],
    )(_sc_body)
    partials = call(ana_scores, gidx.reshape(N))

    loss = pl.pallas_call(
        _tc_comb_body,
        out_shape=jax.ShapeDtypeStruct((1, 1), jnp.float32),
    )(partials, antef, eps_scores.reshape(N, 1))
    return loss[0, 0]


def kernel(eps_scores, ana_scores, solution_mask):
    return _run(eps_scores, ana_scores, solution_mask)


# speedup vs baseline: 1.0334x; 1.0334x over previous
import functools

import jax
import jax.numpy as jnp
from jax import lax
from jax.experimental import pallas as pl
from jax.experimental.pallas import tpu as pltpu
from jax.experimental.pallas import tpu_sc as plsc

N = 2048
NUM_WORKERS = 32
GR = 8
GROUPS = 8
CB = 512
ANA_BUF = 16576
ANA_LEN = N * (N - 1) // 2
NEG = -1e30

FALSE_NEW = 1.2
FALSE_LINK = 0.5
WRONG_LINK = 1.0


def _tc_ante_body(mask_ref, ante_ref, gidx_ref):
    g = pl.program_id(0)
    colv = (lax.broadcasted_iota(jnp.int32, (N, CB), 1) + g * CB).astype(jnp.float32)
    part = jnp.sum(mask_ref[...] * colv, axis=1, keepdims=True)

    @pl.when(g == 0)
    def _():
        ante_ref[...] = part

    @pl.when(g > 0)
    def _():
        ante_ref[...] = ante_ref[...] + part

    @pl.when(g == pl.num_programs(0) - 1)
    def _():
        rowv = lax.broadcasted_iota(jnp.int32, (N, 1), 0).astype(jnp.float32)
        tri = rowv * (rowv - 1.0) * 0.5
        gidx_ref[...] = (tri + ante_ref[...]).astype(jnp.int32)


def _sc_body(ana_hbm, gidx_hbm, out_hbm, idx_t, a0, a1, row8, sa0, sa1):
    c = lax.axis_index("c")
    s = lax.axis_index("s")
    wid = s * 2 + c

    iota = lax.iota(jnp.int32, 16)
    negs16 = jnp.full((16,), NEG, jnp.float32)
    abufs, asems = (a0, a1), (sa0, sa1)

    def group_info(g):
        w = wid if g % 2 == 0 else 31 - wid
        i0 = GR * w + 256 * g
        alen = 2048 * (g + 1)
        start = (i0 * (i0 - 1)) // 2
        start8 = jnp.minimum((start // 8) * 8, ANA_LEN - alen)
        return i0, alen, start8

    for g in range(GROUPS):
        i0, _, _ = group_info(g)
        pltpu.sync_copy(gidx_hbm.at[pl.ds(i0, GR)], idx_t.at[pl.ds(GR * g, GR)])

    def issue(g, p):
        i0, alen, start8 = group_info(g)
        ca = pltpu.async_copy(
            ana_hbm.at[pl.ds(start8, alen)],
            abufs[p].at[pl.ds(0, alen)], asems[p])
        return ca, i0, start8

    pend = issue(0, 0)
    for g in range(GROUPS):
        p = g & 1
        ca, i0, start8 = pend
        ca.wait()
        if g + 1 < GROUPS:
            pend = issue(g + 1, 1 - p)
        ab = abufs[p]
        for k in reversed(range(GR)):
            i = i0 + k
            off = (i * (i - 1)) // 2 - start8
            r = GR * g + k
            q = idx_t[pl.ds((r // 16) * 16, 16)][r % 16] - start8

            bvec = ab[pl.ds(q, 16)]
            ab[pl.ds(q, 16)] = jnp.where(iota == 0, NEG, bvec)
            for h in range(8):
                ab[pl.ds(off + i + 16 * h, 16)] = negs16

            trip = (i + 127) // 128

            def col_body(j, mmax, off=off, ab=ab):
                for h in range(8):
                    mmax = jnp.maximum(mmax, ab[pl.ds(off + j * 128 + h * 16, 16)])
                return mmax

            mmax = lax.fori_loop(0, trip, col_body, negs16)

            row8[k, pl.ds(0, 16)] = mmax
            row8[k, pl.ds(16, 16)] = bvec
        pltpu.sync_copy(row8, out_hbm.at[pl.ds(i0, GR)])


def _tc_comb_body(buf_ref, ante_ref, eps_ref, out_ref):
    m = jnp.max(buf_ref[:, 0:16], axis=1, keepdims=True)
    bg = buf_ref[:, 16:17]
    ante = ante_ref[...]
    ei = eps_ref[...]
    rowv = lax.broadcasted_iota(jnp.int32, (N, 1), 0).astype(jnp.float32)
    non_ana = ante == rowv
    b = jnp.where(non_ana, ei, bg)
    cc = jnp.where(non_ana, FALSE_LINK, WRONG_LINK)
    dd = jnp.where(non_ana, NEG, FALSE_NEW * (1.0 + ei - b))
    tt = cc * (1.0 + m - b)
    rl = jnp.maximum(0.0, jnp.maximum(dd, tt))
    out_ref[...] = jnp.broadcast_to(jnp.sum(rl), (1, 1))


@jax.jit
def _run(eps_scores, ana_scores, solution_mask):
    antef, gidx = pl.pallas_call(
        _tc_ante_body,
        grid=(N // CB,),
        in_specs=[pl.BlockSpec((N, CB), lambda g: (0, g))],
        out_specs=[pl.BlockSpec((N, 1), lambda g: (0, 0)),
                   pl.BlockSpec((N, 1), lambda g: (0, 0))],
        out_shape=[jax.ShapeDtypeStruct((N, 1), jnp.float32),
                   jax.ShapeDtypeStruct((N, 1), jnp.int32)],
    )(solution_mask)

    mesh = plsc.VectorSubcoreMesh(core_axis_name="c", subcore_axis_name="s")
    call = functools.partial(
        pl.kernel,
        mesh=mesh,
        compiler_params=pltpu.CompilerParams(use_tc_tiling_on_sc=False),
        out_type=jax.ShapeDtypeStruct((N, 32), jnp.float32),
        scratch_types=[
            pltpu.VMEM((64,), jnp.int32),
            pltpu.VMEM((ANA_BUF,), jnp.float32),
            pltpu.VMEM((ANA_BUF,), jnp.float32),
            pltpu.VMEM((GR, 32), jnp.float32),
            pltpu.SemaphoreType.DMA,
            pltpu.SemaphoreType.DMA,
        ],
    )(_sc_body)
    partials = call(ana_scores, gidx.reshape(N))

    loss = pl.pallas_call(
        _tc_comb_body,
        out_shape=jax.ShapeDtypeStruct((1, 1), jnp.float32),
    )(partials, antef, eps_scores.reshape(N, 1))
    return loss[0, 0]


def kernel(eps_scores, ana_scores, solution_mask):
    return _run(eps_scores, ana_scores, solution_mask)


# speedup vs baseline: 1.0658x; 1.0314x over previous
import functools

import jax
import jax.numpy as jnp
from jax import lax
from jax.experimental import pallas as pl
from jax.experimental.pallas import tpu as pltpu
from jax.experimental.pallas import tpu_sc as plsc

N = 2048
NUM_WORKERS = 32
GR = 8
GROUPS = 8
CB = 512
ANA_BUF = 16512
ANA_LEN = N * (N - 1) // 2
NEG = -1e30

FALSE_NEW = 1.2
FALSE_LINK = 0.5
WRONG_LINK = 1.0


def _tc_ante_body(mask_ref, ante_ref, gidx_ref):
    g = pl.program_id(0)
    colv = (lax.broadcasted_iota(jnp.int32, (N, CB), 1) + g * CB).astype(jnp.float32)
    part = jnp.sum(mask_ref[...] * colv, axis=1, keepdims=True)

    @pl.when(g == 0)
    def _():
        ante_ref[...] = part

    @pl.when(g > 0)
    def _():
        ante_ref[...] = ante_ref[...] + part

    @pl.when(g == pl.num_programs(0) - 1)
    def _():
        rowv = lax.broadcasted_iota(jnp.int32, (N, 1), 0).astype(jnp.float32)
        tri = rowv * (rowv - 1.0) * 0.5
        gidx_ref[...] = (tri + ante_ref[...]).astype(jnp.int32)


def _sc_body(ana_hbm, gidx_hbm, out_hbm, idx_t, a0, a1, row8, sa0, sa1):
    c = lax.axis_index("c")
    s = lax.axis_index("s")
    wid = s * 2 + c

    iota = lax.iota(jnp.int32, 16)
    negs16 = jnp.full((16,), NEG, jnp.float32)
    abufs, asems = (a0, a1), (sa0, sa1)

    def group_info(g):
        w = wid if g % 2 == 0 else 31 - wid
        i0 = GR * w + 256 * g
        alen = 2048 * (g + 1)
        start = (i0 * (i0 - 1)) // 2
        start8 = jnp.minimum((start // 8) * 8, ANA_LEN - alen)
        return i0, alen, start8

    for g in range(GROUPS):
        i0, _, _ = group_info(g)
        pltpu.sync_copy(gidx_hbm.at[pl.ds(i0, GR)], idx_t.at[pl.ds(GR * g, GR)])

    def issue(g, p):
        i0, alen, start8 = group_info(g)
        ca = pltpu.async_copy(
            ana_hbm.at[pl.ds(start8, alen)],
            abufs[p].at[pl.ds(0, alen)], asems[p])
        return ca, i0, start8

    pend = issue(0, 0)
    for g in range(GROUPS):
        p = g & 1
        ca, i0, start8 = pend
        ca.wait()
        if g + 1 < GROUPS:
            pend = issue(g + 1, 1 - p)
        ab = abufs[p]
        for k in reversed(range(GR)):
            i = i0 + k
            off = (i * (i - 1)) // 2 - start8
            r = GR * g + k
            q = idx_t[pl.ds((r // 16) * 16, 16)][r % 16] - start8

            bvec = ab[pl.ds(q, 16)]
            ab[pl.ds(q, 16)] = jnp.where(iota == 0, NEG, bvec)
            for h in range(4):
                ab[pl.ds(off + i + 16 * h, 16)] = negs16

            trip = (i + 63) // 64

            def col_body(j, mmax, off=off, ab=ab):
                for h in range(4):
                    mmax = jnp.maximum(mmax, ab[pl.ds(off + j * 64 + h * 16, 16)])
                return mmax

            mmax = lax.fori_loop(0, trip, col_body, negs16)

            row8[k, pl.ds(0, 16)] = mmax
            row8[k, pl.ds(16, 16)] = bvec
        pltpu.sync_copy(row8, out_hbm.at[pl.ds(i0, GR)])


def _tc_comb_body(buf_ref, ante_ref, eps_ref, out_ref):
    m = jnp.max(buf_ref[:, 0:16], axis=1, keepdims=True)
    bg = buf_ref[:, 16:17]
    ante = ante_ref[...]
    ei = eps_ref[...]
    rowv = lax.broadcasted_iota(jnp.int32, (N, 1), 0).astype(jnp.float32)
    non_ana = ante == rowv
    b = jnp.where(non_ana, ei, bg)
    cc = jnp.where(non_ana, FALSE_LINK, WRONG_LINK)
    dd = jnp.where(non_ana, NEG, FALSE_NEW * (1.0 + ei - b))
    tt = cc * (1.0 + m - b)
    rl = jnp.maximum(0.0, jnp.maximum(dd, tt))
    out_ref[...] = jnp.broadcast_to(jnp.sum(rl), (1, 1))


@jax.jit
def _run(eps_scores, ana_scores, solution_mask):
    antef, gidx = pl.pallas_call(
        _tc_ante_body,
        grid=(N // CB,),
        in_specs=[pl.BlockSpec((N, CB), lambda g: (0, g))],
        out_specs=[pl.BlockSpec((N, 1), lambda g: (0, 0)),
                   pl.BlockSpec((N, 1), lambda g: (0, 0))],
        out_shape=[jax.ShapeDtypeStruct((N, 1), jnp.float32),
                   jax.ShapeDtypeStruct((N, 1), jnp.int32)],
    )(solution_mask)

    mesh = plsc.VectorSubcoreMesh(core_axis_name="c", subcore_axis_name="s")
    call = functools.partial(
        pl.kernel,
        mesh=mesh,
        compiler_params=pltpu.CompilerParams(use_tc_tiling_on_sc=False),
        out_type=jax.ShapeDtypeStruct((N, 32), jnp.float32),
        scratch_types=[
            pltpu.VMEM((64,), jnp.int32),
            pltpu.VMEM((ANA_BUF,), jnp.float32),
            pltpu.VMEM((ANA_BUF,), jnp.float32),
            pltpu.VMEM((GR, 32), jnp.float32),
            pltpu.SemaphoreType.DMA,
            pltpu.SemaphoreType.DMA,
        ],
    )(_sc_body)
    partials = call(ana_scores, gidx.reshape(N))

    loss = pl.pallas_call(
        _tc_comb_body,
        out_shape=jax.ShapeDtypeStruct((1, 1), jnp.float32),
    )(partials, antef, eps_scores.reshape(N, 1))
    return loss[0, 0]


def kernel(eps_scores, ana_scores, solution_mask):
    return _run(eps_scores, ana_scores, solution_mask)


# speedup vs baseline: 1.0694x; 1.0033x over previous
import functools

import jax
import jax.numpy as jnp
from jax import lax
from jax.experimental import pallas as pl
from jax.experimental.pallas import tpu as pltpu
from jax.experimental.pallas import tpu_sc as plsc

N = 2048
NUM_WORKERS = 32
GR = 8
GROUPS = 8
CB = 1024
ANA_BUF = 16512
ANA_LEN = N * (N - 1) // 2
NEG = -1e30

FALSE_NEW = 1.2
FALSE_LINK = 0.5
WRONG_LINK = 1.0


def _tc_ante_body(mask_ref, ante_ref, gidx_ref):
    g = pl.program_id(0)
    colv = (lax.broadcasted_iota(jnp.int32, (N, CB), 1) + g * CB).astype(jnp.float32)
    part = jnp.sum(mask_ref[...] * colv, axis=1, keepdims=True)

    @pl.when(g == 0)
    def _():
        ante_ref[...] = part

    @pl.when(g > 0)
    def _():
        ante_ref[...] = ante_ref[...] + part

    @pl.when(g == pl.num_programs(0) - 1)
    def _():
        rowv = lax.broadcasted_iota(jnp.int32, (N, 1), 0).astype(jnp.float32)
        tri = rowv * (rowv - 1.0) * 0.5
        gidx_ref[...] = (tri + ante_ref[...]).astype(jnp.int32)


def _sc_body(ana_hbm, gidx_hbm, out_hbm, idx_t, a0, a1, row8, sa0, sa1):
    c = lax.axis_index("c")
    s = lax.axis_index("s")
    wid = s * 2 + c

    iota = lax.iota(jnp.int32, 16)
    negs16 = jnp.full((16,), NEG, jnp.float32)
    abufs, asems = (a0, a1), (sa0, sa1)

    def group_info(g):
        w = wid if g % 2 == 0 else 31 - wid
        i0 = GR * w + 256 * g
        alen = 2048 * (g + 1)
        start = (i0 * (i0 - 1)) // 2
        start8 = jnp.minimum((start // 8) * 8, ANA_LEN - alen)
        return i0, alen, start8

    for g in range(GROUPS):
        i0, _, _ = group_info(g)
        pltpu.sync_copy(gidx_hbm.at[pl.ds(i0, GR)], idx_t.at[pl.ds(GR * g, GR)])

    def issue(g, p):
        i0, alen, start8 = group_info(g)
        ca = pltpu.async_copy(
            ana_hbm.at[pl.ds(start8, alen)],
            abufs[p].at[pl.ds(0, alen)], asems[p])
        return ca, i0, start8

    pend = issue(0, 0)
    for g in range(GROUPS):
        p = g & 1
        ca, i0, start8 = pend
        ca.wait()
        if g + 1 < GROUPS:
            pend = issue(g + 1, 1 - p)
        ab = abufs[p]
        for k in reversed(range(GR)):
            i = i0 + k
            off = (i * (i - 1)) // 2 - start8
            r = GR * g + k
            q = idx_t[pl.ds((r // 16) * 16, 16)][r % 16] - start8

            bvec = ab[pl.ds(q, 16)]
            ab[pl.ds(q, 16)] = jnp.where(iota == 0, NEG, bvec)
            for h in range(4):
                ab[pl.ds(off + i + 16 * h, 16)] = negs16

            trip = (i + 63) // 64

            def col_body(j, mmax, off=off, ab=ab):
                for h in range(4):
                    mmax = jnp.maximum(mmax, ab[pl.ds(off + j * 64 + h * 16, 16)])
                return mmax

            mmax = lax.fori_loop(0, trip, col_body, negs16)

            row8[k, pl.ds(0, 16)] = mmax
            row8[k, pl.ds(16, 16)] = bvec
        pltpu.sync_copy(row8, out_hbm.at[pl.ds(i0, GR)])


def _tc_comb_body(buf_ref, ante_ref, eps_ref, out_ref):
    m = jnp.max(buf_ref[:, 0:16], axis=1, keepdims=True)
    bg = buf_ref[:, 16:17]
    ante = ante_ref[...]
    ei = eps_ref[...]
    rowv = lax.broadcasted_iota(jnp.int32, (N, 1), 0).astype(jnp.float32)
    non_ana = ante == rowv
    b = jnp.where(non_ana, ei, bg)
    cc = jnp.where(non_ana, FALSE_LINK, WRONG_LINK)
    dd = jnp.where(non_ana, NEG, FALSE_NEW * (1.0 + ei - b))
    tt = cc * (1.0 + m - b)
    rl = jnp.maximum(0.0, jnp.maximum(dd, tt))
    out_ref[...] = jnp.broadcast_to(jnp.sum(rl), (1, 1))


@jax.jit
def _run(eps_scores, ana_scores, solution_mask):
    antef, gidx = pl.pallas_call(
        _tc_ante_body,
        grid=(N // CB,),
        in_specs=[pl.BlockSpec((N, CB), lambda g: (0, g))],
        out_specs=[pl.BlockSpec((N, 1), lambda g: (0, 0)),
                   pl.BlockSpec((N, 1), lambda g: (0, 0))],
        out_shape=[jax.ShapeDtypeStruct((N, 1), jnp.float32),
                   jax.ShapeDtypeStruct((N, 1), jnp.int32)],
    )(solution_mask)

    mesh = plsc.VectorSubcoreMesh(core_axis_name="c", subcore_axis_name="s")
    call = functools.partial(
        pl.kernel,
        mesh=mesh,
        compiler_params=pltpu.CompilerParams(use_tc_tiling_on_sc=False),
        out_type=jax.ShapeDtypeStruct((N, 32), jnp.float32),
        scratch_types=[
            pltpu.VMEM((64,), jnp.int32),
            pltpu.VMEM((ANA_BUF,), jnp.float32),
            pltpu.VMEM((ANA_BUF,), jnp.float32),
            pltpu.VMEM((GR, 32), jnp.float32),
            pltpu.SemaphoreType.DMA,
            pltpu.SemaphoreType.DMA,
        ],
    )(_sc_body)
    partials = call(ana_scores, gidx.reshape(N))

    loss = pl.pallas_call(
        _tc_comb_body,
        out_shape=jax.ShapeDtypeStruct((1, 1), jnp.float32),
    )(partials, antef, eps_scores.reshape(N, 1))
    return loss[0, 0]


def kernel(eps_scores, ana_scores, solution_mask):
    return _run(eps_scores, ana_scores, solution_mask)
